# baseline (speedup 1.0000x reference)
import numpy as np
import jax
import jax.numpy as jnp
from jax.experimental import pallas as pl
from jax.experimental.pallas import tpu as pltpu

_B = 1024
_BB = 512
_NEG = -1e30


def _moe_body(x_ref, M1_ref, b1_ref, M2_ref, b2_ref, Wg_ref, bg_ref,
              We1_ref, be1_ref, We2_ref, be2_ref, Ws_ref, bs_ref, out_ref):
    f32 = jnp.float32
    Bb = x_ref.shape[0]
    x = x_ref[...]
    z4 = jnp.zeros((Bb, 4), dtype=f32)

    X6 = jnp.stack(
        [jnp.concatenate(
            [t for r in range(6) for t in (x[:, 2 * y2 + r, :], z4)],
            axis=-1)
         for y2 in range(12)], axis=0)
    Y1 = jax.lax.dot_general(
        X6.reshape(12 * Bb, 192), M1_ref[...],
        (((1,), (0,)), ((), ())), preferred_element_type=f32)
    Y1 = jnp.maximum(Y1 + b1_ref[...], 0.0)

    t = jnp.maximum(Y1[:, :384], Y1[:, 384:])
    pooled = jnp.maximum(t[:, :192], t[:, 192:384])
    pooled = pooled.reshape(12, Bb, 192)

    Y2 = b2_ref[...]
    for dy in range(3):
        Y2 = Y2 + jax.lax.dot_general(
            pooled[dy:dy + 10].reshape(10 * Bb, 192), M2_ref[dy],
            (((1,), (0,)), ((), ())), preferred_element_type=f32)
    Y2 = jnp.maximum(Y2, 0.0)
    H3 = Y2.reshape(10, Bb, 320)

    g = bg_ref[...]
    accs = [be1_ref[e:e + 1, :] for e in range(5)]
    for y in range(10):
        hy = H3[y]
        g = g + jax.lax.dot_general(
            hy, Wg_ref[:, y].reshape(320, 5),
            (((1,), (0,)), ((), ())), preferred_element_type=f32)
        We1_y = We1_ref[:, :, y].reshape(5, 320, 128)
        for e in range(5):
            accs[e] = accs[e] + jax.lax.dot_general(
                hy, We1_y[e], (((1,), (0,)), ((), ())),
                preferred_element_type=f32)
    ehs = [jnp.tanh(a) for a in accs]

    m = g
    vs, ohs = [], []
    for _ in range(3):
        v = jnp.max(m, axis=1, keepdims=True)
        eqf = jnp.where(m >= v, 1.0, 0.0)
        notbefore = jnp.ones((Bb, 1), dtype=f32)
        cols = []
        for e in range(5):
            cur = eqf[:, e:e + 1] * notbefore
            cols.append(cur)
            notbefore = notbefore * (1.0 - eqf[:, e:e + 1])
        oh = jnp.concatenate(cols, axis=1)
        vs.append(v)
        ohs.append(oh)
        m = m + oh * _NEG
    es = [jnp.exp(v - vs[0]) for v in vs]
    denom = es[0] + es[1] + es[2]
    gates = (ohs[0] * es[0] + ohs[1] * es[1] + ohs[2] * es[2]) / denom

    moe = jnp.zeros((Bb, 128), dtype=f32)
    for e in range(5):
        eo = jnp.tanh(
            jax.lax.dot_general(ehs[e], We2_ref[e],
                                (((1,), (0,)), ((), ())),
                                preferred_element_type=f32)
            + be2_ref[e:e + 1, :])
        moe = moe + gates[:, e:e + 1] * eo

    logits = jax.lax.dot_general(moe, Ws_ref[...], (((1,), (0,)), ((), ())),
                                 preferred_element_type=f32) + bs_ref[...]
    mx = jnp.max(logits, axis=1, keepdims=True)
    ex = jnp.exp(logits - mx)
    out_ref[...] = ex / jnp.sum(ex, axis=1, keepdims=True)


def _band_matrices(W1, W2):
    f32 = jnp.float32
    a1 = jnp.concatenate([W1[:, 0], jnp.zeros((16, 5, 24), f32)], axis=-1)
    T1 = jnp.tile(a1, (1, 1, 24))[:, :, :672].reshape(16, 5, 24, 28)
    T1 = T1.reshape(16, 5, 12, 2, 28)
    z = jnp.zeros((16, 1, 12, 2, 28), f32)
    R = jnp.stack([jnp.concatenate([T1, z], axis=1),
                   jnp.concatenate([z, T1], axis=1)], axis=1)
    R = jnp.pad(R, ((0, 0), (0, 0), (0, 0), (0, 0), (0, 0), (0, 4)))
    M1 = R.transpose(2, 5, 1, 4, 3, 0).reshape(192, 768)

    a2 = jnp.concatenate([W2, jnp.zeros((32, 16, 3, 10), f32)], axis=-1)
    T2 = jnp.tile(a2, (1, 1, 1, 10))[:, :, :, :120].reshape(32, 16, 3, 10, 12)
    M2 = T2.transpose(2, 4, 1, 0, 3).reshape(3, 192, 320)
    return M1, M2


def kernel(x, W1, b1, W2, b2, Wg, bg, We1, be1, We2, be2, Ws, bs):
    f32 = jnp.float32
    xs = x.reshape(_B, 28, 28)
    M1, M2 = _band_matrices(W1, W2)
    b1rep = jnp.tile(b1, 48).reshape(1, 768)
    b2rep = jnp.repeat(b2, 10).reshape(1, 320)
    Wg4 = Wg.reshape(32, 10, 10, 5)
    We1r = We1.reshape(5, 32, 10, 10, 128)
    bg2 = bg.reshape(1, 5)
    bs2 = bs.reshape(1, 10)

    out = pl.pallas_call(
        _moe_body,
        grid=(_B // _BB,),
        in_specs=[
            pl.BlockSpec((_BB, 28, 28), lambda i: (i, 0, 0)),
            pl.BlockSpec((192, 768), lambda i: (0, 0)),
            pl.BlockSpec((1, 768), lambda i: (0, 0)),
            pl.BlockSpec((3, 192, 320), lambda i: (0, 0, 0)),
            pl.BlockSpec((1, 320), lambda i: (0, 0)),
            pl.BlockSpec((32, 10, 10, 5), lambda i: (0, 0, 0, 0)),
            pl.BlockSpec((1, 5), lambda i: (0, 0)),
            pl.BlockSpec((5, 32, 10, 10, 128), lambda i: (0, 0, 0, 0, 0)),
            pl.BlockSpec((5, 128), lambda i: (0, 0)),
            pl.BlockSpec((5, 128, 128), lambda i: (0, 0, 0)),
            pl.BlockSpec((5, 128), lambda i: (0, 0)),
            pl.BlockSpec((128, 10), lambda i: (0, 0)),
            pl.BlockSpec((1, 10), lambda i: (0, 0)),
        ],
        out_specs=pl.BlockSpec((_BB, 10), lambda i: (i, 0)),
        out_shape=jax.ShapeDtypeStruct((_B, 10), f32),
        compiler_params=pltpu.CompilerParams(
            dimension_semantics=("arbitrary",)),
    )(xs, M1, b1rep, M2, b2rep, Wg4, bg2, We1r, be1, We2, be2, Ws, bs2)
    return out


# speedup vs baseline: 1.1557x; 1.1557x over previous
import numpy as np
import jax
import jax.numpy as jnp
from jax.experimental import pallas as pl
from jax.experimental.pallas import tpu as pltpu

_B = 1024
_BB = 256
_NEG = -1e30


def _moe_body(x_ref, M1_ref, b1_ref, M2_ref, b2_ref, Wg_ref, bg_ref,
              We1_ref, be1_ref, We2_ref, be2_ref, Ws_ref, bs_ref, out_ref):
    f32 = jnp.float32
    Bb = x_ref.shape[0]
    x = x_ref[...]
    z4 = jnp.zeros((Bb, 4), dtype=f32)

    X6 = jnp.stack(
        [jnp.concatenate(
            [t for r in range(6) for t in (x[:, 2 * y2 + r, :], z4)],
            axis=-1)
         for y2 in range(12)], axis=0)
    Y1 = jax.lax.dot_general(
        X6.reshape(12 * Bb, 192), M1_ref[...],
        (((1,), (0,)), ((), ())), preferred_element_type=f32)
    Y1 = jnp.maximum(Y1 + b1_ref[...], 0.0)

    t = jnp.maximum(Y1[:, :384], Y1[:, 384:])
    pooled = jnp.maximum(t[:, :192], t[:, 192:384])
    pooled = pooled.reshape(12, Bb, 192)

    Y2 = b2_ref[...]
    for dy in range(3):
        Y2 = Y2 + jax.lax.dot_general(
            pooled[dy:dy + 10].reshape(10 * Bb, 192), M2_ref[dy],
            (((1,), (0,)), ((), ())), preferred_element_type=f32)
    Y2 = jnp.maximum(Y2, 0.0)
    H3 = Y2.reshape(10, Bb, 320)

    g = bg_ref[...]
    accs = [be1_ref[e:e + 1, :] for e in range(5)]
    H3b = H3.astype(jnp.bfloat16)
    for y in range(10):
        hy = H3[y]
        g = g + jax.lax.dot_general(
            hy, Wg_ref[:, y].reshape(320, 5),
            (((1,), (0,)), ((), ())), preferred_element_type=f32)
        We1_y = We1_ref[:, :, y].reshape(5, 320, 128)
        for e in range(5):
            accs[e] = accs[e] + jax.lax.dot_general(
                H3b[y], We1_y[e], (((1,), (0,)), ((), ())),
                preferred_element_type=f32)
    ehs = [jnp.tanh(a) for a in accs]

    m = g
    vs, ohs = [], []
    for _ in range(3):
        v = jnp.max(m, axis=1, keepdims=True)
        eqf = jnp.where(m >= v, 1.0, 0.0)
        notbefore = jnp.ones((Bb, 1), dtype=f32)
        cols = []
        for e in range(5):
            cur = eqf[:, e:e + 1] * notbefore
            cols.append(cur)
            notbefore = notbefore * (1.0 - eqf[:, e:e + 1])
        oh = jnp.concatenate(cols, axis=1)
        vs.append(v)
        ohs.append(oh)
        m = m + oh * _NEG
    es = [jnp.exp(v - vs[0]) for v in vs]
    denom = es[0] + es[1] + es[2]
    gates = (ohs[0] * es[0] + ohs[1] * es[1] + ohs[2] * es[2]) / denom

    moe = jnp.zeros((Bb, 128), dtype=f32)
    for e in range(5):
        eo = jnp.tanh(
            jax.lax.dot_general(ehs[e], We2_ref[e],
                                (((1,), (0,)), ((), ())),
                                preferred_element_type=f32)
            + be2_ref[e:e + 1, :])
        moe = moe + gates[:, e:e + 1] * eo

    logits = jax.lax.dot_general(moe, Ws_ref[...], (((1,), (0,)), ((), ())),
                                 preferred_element_type=f32) + bs_ref[...]
    mx = jnp.max(logits, axis=1, keepdims=True)
    ex = jnp.exp(logits - mx)
    out_ref[...] = ex / jnp.sum(ex, axis=1, keepdims=True)


def _band_matrices(W1, W2):
    f32 = jnp.float32
    a1 = jnp.concatenate([W1[:, 0], jnp.zeros((16, 5, 24), f32)], axis=-1)
    T1 = jnp.tile(a1, (1, 1, 24))[:, :, :672].reshape(16, 5, 24, 28)
    T1 = T1.reshape(16, 5, 12, 2, 28)
    z = jnp.zeros((16, 1, 12, 2, 28), f32)
    R = jnp.stack([jnp.concatenate([T1, z], axis=1),
                   jnp.concatenate([z, T1], axis=1)], axis=1)
    R = jnp.pad(R, ((0, 0), (0, 0), (0, 0), (0, 0), (0, 0), (0, 4)))
    M1 = R.transpose(2, 5, 1, 4, 3, 0).reshape(192, 768)

    a2 = jnp.concatenate([W2, jnp.zeros((32, 16, 3, 10), f32)], axis=-1)
    T2 = jnp.tile(a2, (1, 1, 1, 10))[:, :, :, :120].reshape(32, 16, 3, 10, 12)
    M2 = T2.transpose(2, 4, 1, 0, 3).reshape(3, 192, 320)
    return M1, M2


def kernel(x, W1, b1, W2, b2, Wg, bg, We1, be1, We2, be2, Ws, bs):
    f32 = jnp.float32
    xs = x.reshape(_B, 28, 28)
    M1, M2 = _band_matrices(W1, W2)
    b1rep = jnp.tile(b1, 48).reshape(1, 768)
    b2rep = jnp.repeat(b2, 10).reshape(1, 320)
    Wg4 = Wg.reshape(32, 10, 10, 5)
    We1r = We1.reshape(5, 32, 10, 10, 128).astype(jnp.bfloat16)
    bg2 = bg.reshape(1, 5)
    bs2 = bs.reshape(1, 10)

    out = pl.pallas_call(
        _moe_body,
        grid=(_B // _BB,),
        in_specs=[
            pl.BlockSpec((_BB, 28, 28), lambda i: (i, 0, 0)),
            pl.BlockSpec((192, 768), lambda i: (0, 0)),
            pl.BlockSpec((1, 768), lambda i: (0, 0)),
            pl.BlockSpec((3, 192, 320), lambda i: (0, 0, 0)),
            pl.BlockSpec((1, 320), lambda i: (0, 0)),
            pl.BlockSpec((32, 10, 10, 5), lambda i: (0, 0, 0, 0)),
            pl.BlockSpec((1, 5), lambda i: (0, 0)),
            pl.BlockSpec((5, 32, 10, 10, 128), lambda i: (0, 0, 0, 0, 0)),
            pl.BlockSpec((5, 128), lambda i: (0, 0)),
            pl.BlockSpec((5, 128, 128), lambda i: (0, 0, 0)),
            pl.BlockSpec((5, 128), lambda i: (0, 0)),
            pl.BlockSpec((128, 10), lambda i: (0, 0)),
            pl.BlockSpec((1, 10), lambda i: (0, 0)),
        ],
        out_specs=pl.BlockSpec((_BB, 10), lambda i: (i, 0)),
        out_shape=jax.ShapeDtypeStruct((_B, 10), f32),
        compiler_params=pltpu.CompilerParams(
            dimension_semantics=("arbitrary",)),
    )(xs, M1, b1rep, M2, b2rep, Wg4, bg2, We1r, be1, We2, be2, Ws, bs2)
    return out


# speedup vs baseline: 1.2300x; 1.0644x over previous
import numpy as np
import jax
import jax.numpy as jnp
from jax.experimental import pallas as pl
from jax.experimental.pallas import tpu as pltpu

_B = 1024
_BB = 256
_NEG = -1e30


def _moe_body(x_ref, M1_ref, b1_ref, M2_ref, b2_ref, Wg_ref, bg_ref,
              We1_ref, be1_ref, We2_ref, be2_ref, Ws_ref, bs_ref, out_ref):
    f32 = jnp.float32
    Bb = x_ref.shape[1]
    x = x_ref[...]
    z4 = jnp.zeros((Bb, 4), dtype=f32)

    X6 = jnp.stack(
        [jnp.concatenate(
            [t for r in range(6) for t in (x[2 * y2 + r], z4)],
            axis=-1)
         for y2 in range(12)], axis=0)
    Y1 = jax.lax.dot_general(
        X6.reshape(12 * Bb, 192), M1_ref[...],
        (((1,), (0,)), ((), ())), preferred_element_type=f32)
    Y1 = jnp.maximum(Y1 + b1_ref[...], 0.0)

    t = jnp.maximum(Y1[:, :384], Y1[:, 384:])
    pooled = jnp.maximum(t[:, :192], t[:, 192:384])
    pooled = pooled.reshape(12, Bb, 192)

    Y2 = b2_ref[...]
    for dy in range(3):
        Y2 = Y2 + jax.lax.dot_general(
            pooled[dy:dy + 10].reshape(10 * Bb, 192), M2_ref[dy],
            (((1,), (0,)), ((), ())), preferred_element_type=f32)
    Y2 = jnp.maximum(Y2, 0.0)
    H3 = Y2.reshape(10, Bb, 320)

    g = bg_ref[...]
    accs = [be1_ref[e:e + 1, :] for e in range(5)]
    H3b = H3.astype(jnp.bfloat16)
    for y in range(10):
        g = g + jax.lax.dot_general(
            H3[y], Wg_ref[y], (((1,), (0,)), ((), ())),
            preferred_element_type=f32)
        for e in range(5):
            accs[e] = accs[e] + jax.lax.dot_general(
                H3b[y], We1_ref[y, e], (((1,), (0,)), ((), ())),
                preferred_element_type=f32)
    ehs = [jnp.tanh(a) for a in accs]

    m = g
    vs, ohs = [], []
    for _ in range(3):
        v = jnp.max(m, axis=1, keepdims=True)
        eqf = jnp.where(m >= v, 1.0, 0.0)
        notbefore = jnp.ones((Bb, 1), dtype=f32)
        cols = []
        for e in range(5):
            cur = eqf[:, e:e + 1] * notbefore
            cols.append(cur)
            notbefore = notbefore * (1.0 - eqf[:, e:e + 1])
        oh = jnp.concatenate(cols, axis=1)
        vs.append(v)
        ohs.append(oh)
        m = m + oh * _NEG
    es = [jnp.exp(v - vs[0]) for v in vs]
    denom = es[0] + es[1] + es[2]
    gates = (ohs[0] * es[0] + ohs[1] * es[1] + ohs[2] * es[2]) / denom

    moe = jnp.zeros((Bb, 128), dtype=f32)
    for e in range(5):
        eo = jnp.tanh(
            jax.lax.dot_general(ehs[e], We2_ref[e],
                                (((1,), (0,)), ((), ())),
                                preferred_element_type=f32)
            + be2_ref[e:e + 1, :])
        moe = moe + gates[:, e:e + 1] * eo

    logits = jax.lax.dot_general(moe, Ws_ref[...], (((1,), (0,)), ((), ())),
                                 preferred_element_type=f32) + bs_ref[...]
    mx = jnp.max(logits, axis=1, keepdims=True)
    ex = jnp.exp(logits - mx)
    out_ref[...] = ex / jnp.sum(ex, axis=1, keepdims=True)


def _band_matrices(W1, W2):
    f32 = jnp.float32
    a1 = jnp.concatenate([W1[:, 0], jnp.zeros((16, 5, 24), f32)], axis=-1)
    T1 = jnp.tile(a1, (1, 1, 24))[:, :, :672].reshape(16, 5, 24, 28)
    T1 = T1.reshape(16, 5, 12, 2, 28)
    z = jnp.zeros((16, 1, 12, 2, 28), f32)
    R = jnp.stack([jnp.concatenate([T1, z], axis=1),
                   jnp.concatenate([z, T1], axis=1)], axis=1)
    R = jnp.pad(R, ((0, 0), (0, 0), (0, 0), (0, 0), (0, 0), (0, 4)))
    M1 = R.transpose(2, 5, 1, 4, 3, 0).reshape(192, 768)

    a2 = jnp.concatenate([W2, jnp.zeros((32, 16, 3, 10), f32)], axis=-1)
    T2 = jnp.tile(a2, (1, 1, 1, 10))[:, :, :, :120].reshape(32, 16, 3, 10, 12)
    M2 = T2.transpose(2, 4, 1, 0, 3).reshape(3, 192, 320)
    return M1, M2


def kernel(x, W1, b1, W2, b2, Wg, bg, We1, be1, We2, be2, Ws, bs):
    f32 = jnp.float32
    xT = x.reshape(_B, 28, 28).transpose(1, 0, 2)
    M1, M2 = _band_matrices(W1, W2)
    b1rep = jnp.tile(b1, 48).reshape(1, 768)
    b2rep = jnp.repeat(b2, 10).reshape(1, 320)
    We1p = We1.astype(jnp.bfloat16).reshape(5, 32, 10, 10, 128) \
        .transpose(2, 0, 1, 3, 4).reshape(10, 5, 320, 128)
    Wgp = Wg.reshape(32, 10, 10, 5).transpose(1, 0, 2, 3).reshape(10, 320, 5)
    bg2 = bg.reshape(1, 5)
    bs2 = bs.reshape(1, 10)

    out = pl.pallas_call(
        _moe_body,
        grid=(_B // _BB,),
        in_specs=[
            pl.BlockSpec((28, _BB, 28), lambda i: (0, i, 0)),
            pl.BlockSpec((192, 768), lambda i: (0, 0)),
            pl.BlockSpec((1, 768), lambda i: (0, 0)),
            pl.BlockSpec((3, 192, 320), lambda i: (0, 0, 0)),
            pl.BlockSpec((1, 320), lambda i: (0, 0)),
            pl.BlockSpec((10, 320, 5), lambda i: (0, 0, 0)),
            pl.BlockSpec((1, 5), lambda i: (0, 0)),
            pl.BlockSpec((10, 5, 320, 128), lambda i: (0, 0, 0, 0)),
            pl.BlockSpec((5, 128), lambda i: (0, 0)),
            pl.BlockSpec((5, 128, 128), lambda i: (0, 0, 0)),
            pl.BlockSpec((5, 128), lambda i: (0, 0)),
            pl.BlockSpec((128, 10), lambda i: (0, 0)),
            pl.BlockSpec((1, 10), lambda i: (0, 0)),
        ],
        out_specs=pl.BlockSpec((_BB, 10), lambda i: (i, 0)),
        out_shape=jax.ShapeDtypeStruct((_B, 10), f32),
        compiler_params=pltpu.CompilerParams(
            dimension_semantics=("arbitrary",)),
    )(xT, M1, b1rep, M2, b2rep, Wgp, bg2, We1p, be1, We2, be2, Ws, bs2)
    return out


# speedup vs baseline: 1.2854x; 1.0450x over previous
import numpy as np
import jax
import jax.numpy as jnp
from jax.experimental import pallas as pl
from jax.experimental.pallas import tpu as pltpu

_B = 1024
_BB = 256
_NEG = -1e30


def _moe_body(x_ref, M1_ref, b1_ref, M2_ref, b2_ref, Wg_ref, bg_ref,
              We1_ref, be1_ref, We2_ref, be2_ref, Ws_ref, bs_ref, out_ref):
    f32 = jnp.float32
    bf16 = jnp.bfloat16
    Bb = x_ref.shape[1]
    x = x_ref[...]
    z4 = jnp.zeros((Bb, 4), dtype=bf16)

    X6 = jnp.stack(
        [jnp.concatenate(
            [t for r in range(6) for t in (x[2 * y2 + r], z4)],
            axis=-1)
         for y2 in range(12)], axis=0)
    Y1 = jax.lax.dot_general(
        X6.reshape(12 * Bb, 192), M1_ref[...],
        (((1,), (0,)), ((), ())), preferred_element_type=f32)
    Y1 = jnp.maximum(Y1.astype(bf16) + b1_ref[...], 0.0)

    t = jnp.maximum(Y1[:, :384], Y1[:, 384:])
    pooled = jnp.maximum(t[:, :192], t[:, 192:384])
    pooled = pooled.reshape(12, Bb, 192)

    Y2 = jnp.zeros((10 * Bb, 320), dtype=f32)
    for dy in range(3):
        Y2 = Y2 + jax.lax.dot_general(
            pooled[dy:dy + 10].reshape(10 * Bb, 192), M2_ref[dy],
            (((1,), (0,)), ((), ())), preferred_element_type=f32)
    Y2 = jnp.maximum(Y2.astype(bf16) + b2_ref[...], 0.0)
    H3 = Y2.reshape(10, Bb, 320)

    g = bg_ref[...]
    accs = [be1_ref[e:e + 1, :] for e in range(5)]
    for y in range(10):
        g = g + jax.lax.dot_general(
            H3[y], Wg_ref[y], (((1,), (0,)), ((), ())),
            preferred_element_type=f32)
        for e in range(5):
            accs[e] = accs[e] + jax.lax.dot_general(
                H3[y], We1_ref[y, e], (((1,), (0,)), ((), ())),
                preferred_element_type=f32)
    ehs = [jnp.tanh(a) for a in accs]

    m = g
    vs, ohs = [], []
    for _ in range(3):
        v = jnp.max(m, axis=1, keepdims=True)
        eqf = jnp.where(m >= v, 1.0, 0.0)
        notbefore = jnp.ones((Bb, 1), dtype=f32)
        cols = []
        for e in range(5):
            cur = eqf[:, e:e + 1] * notbefore
            cols.append(cur)
            notbefore = notbefore * (1.0 - eqf[:, e:e + 1])
        oh = jnp.concatenate(cols, axis=1)
        vs.append(v)
        ohs.append(oh)
        m = m + oh * _NEG
    es = [jnp.exp(v - vs[0]) for v in vs]
    denom = es[0] + es[1] + es[2]
    gates = (ohs[0] * es[0] + ohs[1] * es[1] + ohs[2] * es[2]) / denom

    moe = jnp.zeros((Bb, 128), dtype=f32)
    for e in range(5):
        eo = jnp.tanh(
            jax.lax.dot_general(ehs[e].astype(bf16), We2_ref[e],
                                (((1,), (0,)), ((), ())),
                                preferred_element_type=f32)
            + be2_ref[e:e + 1, :])
        moe = moe + gates[:, e:e + 1] * eo

    logits = jax.lax.dot_general(moe, Ws_ref[...], (((1,), (0,)), ((), ())),
                                 preferred_element_type=f32) + bs_ref[...]
    mx = jnp.max(logits, axis=1, keepdims=True)
    ex = jnp.exp(logits - mx)
    out_ref[...] = ex / jnp.sum(ex, axis=1, keepdims=True)


def _band_matrices(W1, W2):
    f32 = jnp.float32
    a1 = jnp.concatenate([W1[:, 0], jnp.zeros((16, 5, 24), f32)], axis=-1)
    T1 = jnp.tile(a1, (1, 1, 24))[:, :, :672].reshape(16, 5, 24, 28)
    T1 = T1.reshape(16, 5, 12, 2, 28)
    z = jnp.zeros((16, 1, 12, 2, 28), f32)
    R = jnp.stack([jnp.concatenate([T1, z], axis=1),
                   jnp.concatenate([z, T1], axis=1)], axis=1)
    R = jnp.pad(R, ((0, 0), (0, 0), (0, 0), (0, 0), (0, 0), (0, 4)))
    M1 = R.transpose(2, 5, 1, 4, 3, 0).reshape(192, 768)

    a2 = jnp.concatenate([W2, jnp.zeros((32, 16, 3, 10), f32)], axis=-1)
    T2 = jnp.tile(a2, (1, 1, 1, 10))[:, :, :, :120].reshape(32, 16, 3, 10, 12)
    M2 = T2.transpose(2, 4, 1, 0, 3).reshape(3, 192, 320)
    return M1, M2


def kernel(x, W1, b1, W2, b2, Wg, bg, We1, be1, We2, be2, Ws, bs):
    f32 = jnp.float32
    bf16 = jnp.bfloat16
    xT = x.astype(bf16).reshape(_B, 28, 28).transpose(1, 0, 2)
    M1, M2 = _band_matrices(W1, W2)
    M1 = M1.astype(bf16)
    M2 = M2.astype(bf16)
    b1rep = jnp.tile(b1, 48).reshape(1, 768).astype(bf16)
    b2rep = jnp.repeat(b2, 10).reshape(1, 320).astype(bf16)
    We1p = We1.astype(bf16).reshape(5, 32, 10, 10, 128) \
        .transpose(2, 0, 1, 3, 4).reshape(10, 5, 320, 128)
    Wgp = Wg.reshape(32, 10, 10, 5).transpose(1, 0, 2, 3) \
        .reshape(10, 320, 5).astype(bf16)
    We2 = We2.astype(bf16)
    bg2 = bg.reshape(1, 5)
    bs2 = bs.reshape(1, 10)

    out = pl.pallas_call(
        _moe_body,
        grid=(_B // _BB,),
        in_specs=[
            pl.BlockSpec((28, _BB, 28), lambda i: (0, i, 0)),
            pl.BlockSpec((192, 768), lambda i: (0, 0)),
            pl.BlockSpec((1, 768), lambda i: (0, 0)),
            pl.BlockSpec((3, 192, 320), lambda i: (0, 0, 0)),
            pl.BlockSpec((1, 320), lambda i: (0, 0)),
            pl.BlockSpec((10, 320, 5), lambda i: (0, 0, 0)),
            pl.BlockSpec((1, 5), lambda i: (0, 0)),
            pl.BlockSpec((10, 5, 320, 128), lambda i: (0, 0, 0, 0)),
            pl.BlockSpec((5, 128), lambda i: (0, 0)),
            pl.BlockSpec((5, 128, 128), lambda i: (0, 0, 0)),
            pl.BlockSpec((5, 128), lambda i: (0, 0)),
            pl.BlockSpec((128, 10), lambda i: (0, 0)),
            pl.BlockSpec((1, 10), lambda i: (0, 0)),
        ],
        out_specs=pl.BlockSpec((_BB, 10), lambda i: (i, 0)),
        out_shape=jax.ShapeDtypeStruct((_B, 10), f32),
        compiler_params=pltpu.CompilerParams(
            dimension_semantics=("arbitrary",)),
    )(xT, M1, b1rep, M2, b2rep, Wgp, bg2, We1p, be1, We2, be2, Ws, bs2)
    return out


# speedup vs baseline: 1.3978x; 1.0874x over previous
import numpy as np
import jax
import jax.numpy as jnp
from jax.experimental import pallas as pl
from jax.experimental.pallas import tpu as pltpu

_B = 1024
_BB = 256
_NEG = -1e30


def _moe_body(x_ref, M1_ref, b1_ref, M2_ref, b2_ref, Wg_ref, bg_ref,
              We1_ref, be1_ref, We2_ref, be2_ref, Ws_ref, bs_ref, out_ref):
    f32 = jnp.float32
    bf16 = jnp.bfloat16
    Bb = x_ref.shape[1]
    x = x_ref[...]
    z4 = jnp.zeros((Bb, 4), dtype=bf16)

    X6 = jnp.stack(
        [jnp.concatenate(
            [t for r in range(6) for t in (x[2 * y2 + r], z4)],
            axis=-1)
         for y2 in range(12)], axis=0)
    Y1 = jax.lax.dot_general(
        X6.reshape(12 * Bb, 192), M1_ref[...],
        (((1,), (0,)), ((), ())), preferred_element_type=f32)
    Y1 = jnp.maximum(Y1.astype(bf16) + b1_ref[...], 0.0)

    t = jnp.maximum(Y1[:, :384], Y1[:, 384:])
    pooled = jnp.maximum(t[:, :192], t[:, 192:384])
    pooled = pooled.reshape(12, Bb, 192)

    Y2 = jnp.zeros((10 * Bb, 320), dtype=f32)
    for dy in range(3):
        Y2 = Y2 + jax.lax.dot_general(
            pooled[dy:dy + 10].reshape(10 * Bb, 192), M2_ref[dy],
            (((1,), (0,)), ((), ())), preferred_element_type=f32)
    Y2 = jnp.maximum(Y2.astype(bf16) + b2_ref[...], 0.0)
    H3 = Y2.reshape(10, Bb, 320)

    g = bg_ref[...]
    accs = [be1_ref[e:e + 1, :] for e in range(5)]
    for y in range(10):
        g = g + jax.lax.dot_general(
            H3[y], Wg_ref[y], (((1,), (0,)), ((), ())),
            preferred_element_type=f32)
        We1_y = We1_ref[:, :, y].reshape(5, 320, 128)
        for e in range(5):
            accs[e] = accs[e] + jax.lax.dot_general(
                H3[y], We1_y[e], (((1,), (0,)), ((), ())),
                preferred_element_type=f32)
    ehs = [jnp.tanh(a) for a in accs]

    m = g
    vs, ohs = [], []
    for _ in range(3):
        v = jnp.max(m, axis=1, keepdims=True)
        eqf = jnp.where(m >= v, 1.0, 0.0)
        notbefore = jnp.ones((Bb, 1), dtype=f32)
        cols = []
        for e in range(5):
            cur = eqf[:, e:e + 1] * notbefore
            cols.append(cur)
            notbefore = notbefore * (1.0 - eqf[:, e:e + 1])
        oh = jnp.concatenate(cols, axis=1)
        vs.append(v)
        ohs.append(oh)
        m = m + oh * _NEG
    es = [jnp.exp(v - vs[0]) for v in vs]
    denom = es[0] + es[1] + es[2]
    gates = (ohs[0] * es[0] + ohs[1] * es[1] + ohs[2] * es[2]) / denom

    moe = jnp.zeros((Bb, 128), dtype=f32)
    for e in range(5):
        eo = jnp.tanh(
            jax.lax.dot_general(ehs[e].astype(bf16), We2_ref[e],
                                (((1,), (0,)), ((), ())),
                                preferred_element_type=f32)
            + be2_ref[e:e + 1, :])
        moe = moe + gates[:, e:e + 1] * eo

    logits = jax.lax.dot_general(moe, Ws_ref[...], (((1,), (0,)), ((), ())),
                                 preferred_element_type=f32) + bs_ref[...]
    mx = jnp.max(logits, axis=1, keepdims=True)
    ex = jnp.exp(logits - mx)
    out_ref[...] = ex / jnp.sum(ex, axis=1, keepdims=True)


def _band_matrices(W1, W2):
    f32 = jnp.float32
    a1 = jnp.concatenate([W1[:, 0], jnp.zeros((16, 5, 24), f32)], axis=-1)
    T1 = jnp.tile(a1, (1, 1, 24))[:, :, :672].reshape(16, 5, 24, 28)
    T1 = T1.reshape(16, 5, 12, 2, 28)
    z = jnp.zeros((16, 1, 12, 2, 28), f32)
    R = jnp.stack([jnp.concatenate([T1, z], axis=1),
                   jnp.concatenate([z, T1], axis=1)], axis=1)
    R = jnp.pad(R, ((0, 0), (0, 0), (0, 0), (0, 0), (0, 0), (0, 4)))
    M1 = R.transpose(2, 5, 1, 4, 3, 0).reshape(192, 768)

    a2 = jnp.concatenate([W2, jnp.zeros((32, 16, 3, 10), f32)], axis=-1)
    T2 = jnp.tile(a2, (1, 1, 1, 10))[:, :, :, :120].reshape(32, 16, 3, 10, 12)
    M2 = T2.transpose(2, 4, 1, 0, 3).reshape(3, 192, 320)
    return M1, M2


def kernel(x, W1, b1, W2, b2, Wg, bg, We1, be1, We2, be2, Ws, bs):
    f32 = jnp.float32
    bf16 = jnp.bfloat16
    xT = x.astype(bf16).reshape(_B, 28, 28).transpose(1, 0, 2)
    M1, M2 = _band_matrices(W1, W2)
    M1 = M1.astype(bf16)
    M2 = M2.astype(bf16)
    b1rep = jnp.tile(b1, 48).reshape(1, 768).astype(bf16)
    b2rep = jnp.repeat(b2, 10).reshape(1, 320).astype(bf16)
    We1p = We1.astype(bf16).reshape(5, 32, 10, 10, 128)
    Wgp = Wg.reshape(32, 10, 10, 5).transpose(1, 0, 2, 3) \
        .reshape(10, 320, 5).astype(bf16)
    We2 = We2.astype(bf16)
    bg2 = bg.reshape(1, 5)
    bs2 = bs.reshape(1, 10)

    out = pl.pallas_call(
        _moe_body,
        grid=(_B // _BB,),
        in_specs=[
            pl.BlockSpec((28, _BB, 28), lambda i: (0, i, 0)),
            pl.BlockSpec((192, 768), lambda i: (0, 0)),
            pl.BlockSpec((1, 768), lambda i: (0, 0)),
            pl.BlockSpec((3, 192, 320), lambda i: (0, 0, 0)),
            pl.BlockSpec((1, 320), lambda i: (0, 0)),
            pl.BlockSpec((10, 320, 5), lambda i: (0, 0, 0)),
            pl.BlockSpec((1, 5), lambda i: (0, 0)),
            pl.BlockSpec((5, 32, 10, 10, 128), lambda i: (0, 0, 0, 0, 0)),
            pl.BlockSpec((5, 128), lambda i: (0, 0)),
            pl.BlockSpec((5, 128, 128), lambda i: (0, 0, 0)),
            pl.BlockSpec((5, 128), lambda i: (0, 0)),
            pl.BlockSpec((128, 10), lambda i: (0, 0)),
            pl.BlockSpec((1, 10), lambda i: (0, 0)),
        ],
        out_specs=pl.BlockSpec((_BB, 10), lambda i: (i, 0)),
        out_shape=jax.ShapeDtypeStruct((_B, 10), f32),
        compiler_params=pltpu.CompilerParams(
            dimension_semantics=("arbitrary",)),
    )(xT, M1, b1rep, M2, b2rep, Wgp, bg2, We1p, be1, We2, be2, Ws, bs2)
    return out


# speedup vs baseline: 1.4060x; 1.0059x over previous
import numpy as np
import jax
import jax.numpy as jnp
from jax.experimental import pallas as pl
from jax.experimental.pallas import tpu as pltpu

_B = 1024
_BB = 512
_NEG = -1e30


def _moe_body(x_ref, M1_ref, b1_ref, M2_ref, b2_ref, Wg_ref, bg_ref,
              We1_ref, be1_ref, We2_ref, be2_ref, Ws_ref, bs_ref, out_ref):
    f32 = jnp.float32
    bf16 = jnp.bfloat16
    Bb = x_ref.shape[1]
    x = x_ref[...]
    z4 = jnp.zeros((Bb, 4), dtype=bf16)

    X6 = jnp.stack(
        [jnp.concatenate(
            [t for r in range(6) for t in (x[2 * y2 + r], z4)],
            axis=-1)
         for y2 in range(12)], axis=0)
    Y1 = jax.lax.dot_general(
        X6.reshape(12 * Bb, 192), M1_ref[...],
        (((1,), (0,)), ((), ())), preferred_element_type=f32)
    Y1 = jnp.maximum(Y1.astype(bf16) + b1_ref[...], 0.0)

    t = jnp.maximum(Y1[:, :384], Y1[:, 384:])
    pooled = jnp.maximum(t[:, :192], t[:, 192:384])
    pooled = pooled.reshape(12, Bb, 192)

    Y2 = jnp.zeros((10 * Bb, 320), dtype=f32)
    for dy in range(3):
        Y2 = Y2 + jax.lax.dot_general(
            pooled[dy:dy + 10].reshape(10 * Bb, 192), M2_ref[dy],
            (((1,), (0,)), ((), ())), preferred_element_type=f32)
    Y2 = jnp.maximum(Y2.astype(bf16) + b2_ref[...], 0.0)
    H3 = Y2.reshape(10, Bb, 320)

    g = bg_ref[...]
    accs = [be1_ref[e:e + 1, :] for e in range(5)]
    for y in range(10):
        g = g + jax.lax.dot_general(
            H3[y], Wg_ref[y], (((1,), (0,)), ((), ())),
            preferred_element_type=f32)
        We1_y = We1_ref[:, :, y].reshape(5, 320, 128)
        for e in range(5):
            accs[e] = accs[e] + jax.lax.dot_general(
                H3[y], We1_y[e], (((1,), (0,)), ((), ())),
                preferred_element_type=f32)
    ehs = [jnp.tanh(a) for a in accs]

    m = g
    vs, ohs = [], []
    for _ in range(3):
        v = jnp.max(m, axis=1, keepdims=True)
        eqf = jnp.where(m >= v, 1.0, 0.0)
        notbefore = jnp.ones((Bb, 1), dtype=f32)
        cols = []
        for e in range(5):
            cur = eqf[:, e:e + 1] * notbefore
            cols.append(cur)
            notbefore = notbefore * (1.0 - eqf[:, e:e + 1])
        oh = jnp.concatenate(cols, axis=1)
        vs.append(v)
        ohs.append(oh)
        m = m + oh * _NEG
    es = [jnp.exp(v - vs[0]) for v in vs]
    denom = es[0] + es[1] + es[2]
    gates = (ohs[0] * es[0] + ohs[1] * es[1] + ohs[2] * es[2]) / denom

    moe = jnp.zeros((Bb, 128), dtype=f32)
    for e in range(5):
        eo = jnp.tanh(
            jax.lax.dot_general(ehs[e].astype(bf16), We2_ref[e],
                                (((1,), (0,)), ((), ())),
                                preferred_element_type=f32)
            + be2_ref[e:e + 1, :])
        moe = moe + gates[:, e:e + 1] * eo

    logits = jax.lax.dot_general(moe, Ws_ref[...], (((1,), (0,)), ((), ())),
                                 preferred_element_type=f32) + bs_ref[...]
    mx = jnp.max(logits, axis=1, keepdims=True)
    ex = jnp.exp(logits - mx)
    out_ref[...] = ex / jnp.sum(ex, axis=1, keepdims=True)


def _band_matrices(W1, W2):
    f32 = jnp.float32
    a1 = jnp.concatenate([W1[:, 0], jnp.zeros((16, 5, 24), f32)], axis=-1)
    T1 = jnp.tile(a1, (1, 1, 24))[:, :, :672].reshape(16, 5, 24, 28)
    T1 = T1.reshape(16, 5, 12, 2, 28)
    z = jnp.zeros((16, 1, 12, 2, 28), f32)
    R = jnp.stack([jnp.concatenate([T1, z], axis=1),
                   jnp.concatenate([z, T1], axis=1)], axis=1)
    R = jnp.pad(R, ((0, 0), (0, 0), (0, 0), (0, 0), (0, 0), (0, 4)))
    M1 = R.transpose(2, 5, 1, 4, 3, 0).reshape(192, 768)

    a2 = jnp.concatenate([W2, jnp.zeros((32, 16, 3, 10), f32)], axis=-1)
    T2 = jnp.tile(a2, (1, 1, 1, 10))[:, :, :, :120].reshape(32, 16, 3, 10, 12)
    M2 = T2.transpose(2, 4, 1, 0, 3).reshape(3, 192, 320)
    return M1, M2


def kernel(x, W1, b1, W2, b2, Wg, bg, We1, be1, We2, be2, Ws, bs):
    f32 = jnp.float32
    bf16 = jnp.bfloat16
    xT = x.astype(bf16).reshape(_B, 28, 28).transpose(1, 0, 2)
    M1, M2 = _band_matrices(W1, W2)
    M1 = M1.astype(bf16)
    M2 = M2.astype(bf16)
    b1rep = jnp.tile(b1, 48).reshape(1, 768).astype(bf16)
    b2rep = jnp.repeat(b2, 10).reshape(1, 320).astype(bf16)
    We1p = We1.astype(bf16).reshape(5, 32, 10, 10, 128)
    Wgp = Wg.reshape(32, 10, 10, 5).transpose(1, 0, 2, 3) \
        .reshape(10, 320, 5).astype(bf16)
    We2 = We2.astype(bf16)
    bg2 = bg.reshape(1, 5)
    bs2 = bs.reshape(1, 10)

    out = pl.pallas_call(
        _moe_body,
        grid=(_B // _BB,),
        in_specs=[
            pl.BlockSpec((28, _BB, 28), lambda i: (0, i, 0)),
            pl.BlockSpec((192, 768), lambda i: (0, 0)),
            pl.BlockSpec((1, 768), lambda i: (0, 0)),
            pl.BlockSpec((3, 192, 320), lambda i: (0, 0, 0)),
            pl.BlockSpec((1, 320), lambda i: (0, 0)),
            pl.BlockSpec((10, 320, 5), lambda i: (0, 0, 0)),
            pl.BlockSpec((1, 5), lambda i: (0, 0)),
            pl.BlockSpec((5, 32, 10, 10, 128), lambda i: (0, 0, 0, 0, 0)),
            pl.BlockSpec((5, 128), lambda i: (0, 0)),
            pl.BlockSpec((5, 128, 128), lambda i: (0, 0, 0)),
            pl.BlockSpec((5, 128), lambda i: (0, 0)),
            pl.BlockSpec((128, 10), lambda i: (0, 0)),
            pl.BlockSpec((1, 10), lambda i: (0, 0)),
        ],
        out_specs=pl.BlockSpec((_BB, 10), lambda i: (i, 0)),
        out_shape=jax.ShapeDtypeStruct((_B, 10), f32),
        compiler_params=pltpu.CompilerParams(
            dimension_semantics=("arbitrary",)),
    )(xT, M1, b1rep, M2, b2rep, Wgp, bg2, We1p, be1, We2, be2, Ws, bs2)
    return out
